# baseline (speedup 1.0000x reference)
import functools

import jax
import jax.numpy as jnp
from jax import lax
from jax.experimental import pallas as pl
from jax.experimental.pallas import tpu as pltpu
from jax.experimental.pallas import tpu_sc as plsc

EPS = 1e-12
LANES = 16
GT = 8
SCH = 8

_GATHER_DNUMS = lax.GatherDimensionNumbers(
    offset_dims=(), collapsed_slice_dims=(0,), start_index_map=(0,))


def _lane_rotate(x, k):
    idx = jnp.bitwise_and(lax.iota(jnp.int32, LANES) + k, LANES - 1)
    return lax.gather(x, idx[:, None], _GATHER_DNUMS, slice_sizes=(1,),
                      mode=lax.GatherScatterMode.PROMISE_IN_BOUNDS)


def _lane_allsum(x):
    for k in (8, 4, 2, 1):
        x = x + _lane_rotate(x, k)
    return x


def _rsqrt_vec(x):
    i = lax.bitcast_convert_type(x, jnp.int32)
    i = jnp.int32(0x5F3759DF) - lax.shift_right_logical(i, 1)
    y = lax.bitcast_convert_type(i, jnp.float32)
    for _ in range(3):
        y = y * (1.5 - 0.5 * x * y * y)
    return y


@functools.lru_cache(maxsize=None)
def _build(B, S, D):
    info = plsc.get_sparse_core_info()
    NC, NS = info.num_cores, info.num_subcores
    NW = NC * NS
    T = B * S
    per_w = T // NW
    s_per_w = S // NW
    n_chunks = s_per_w // SCH
    CT = B * SCH
    NV = D // LANES
    n_pairs = n_chunks // 2

    mesh = plsc.VectorSubcoreMesh(core_axis_name="c", subcore_axis_name="s")

    @functools.partial(
        pl.kernel,
        mesh=mesh,
        out_type=jax.ShapeDtypeStruct((B, S, D), jnp.float32),
        scratch_types=[
            pltpu.VMEM((per_w,), jnp.int32),
            pltpu.VMEM((CT, D), jnp.float32),
            pltpu.VMEM((CT, D), jnp.float32),
            pltpu.VMEM((CT, D), jnp.float32),
            pltpu.VMEM((CT, D), jnp.float32),
            pltpu.VMEM((SCH, D), jnp.float32),
            pltpu.VMEM((SCH, D), jnp.float32),
            pltpu.VMEM((D,), jnp.float32),
            pltpu.VMEM((D,), jnp.float32),
            pltpu.SemaphoreType.DMA,
            pltpu.SemaphoreType.DMA,
            pltpu.SemaphoreType.DMA,
            pltpu.SemaphoreType.DMA,
        ],
    )
    def embed_ln(ids_hbm, wt_hbm, pt_hbm, g_hbm, b_hbm, out_hbm,
                 idx_all, xb0, xb1, ob0, ob1, pb0, pb1, g_v, b_v,
                 sem0, sem1, osem0, osem1):
        wid = lax.axis_index("s") * NC + lax.axis_index("c")
        s_lo = wid * s_per_w
        pltpu.sync_copy(g_hbm, g_v)
        pltpu.sync_copy(b_hbm, b_v)
        pltpu.sync_copy(ids_hbm.at[pl.ds(wid * per_w, per_w)], idx_all)

        bufs = ((xb0, ob0, pb0, sem0, osem0), (xb1, ob1, pb1, sem1, osem1))

        def issue(c, bi):
            xb, ob, pb, sem, osem = bufs[bi]
            pltpu.async_copy(wt_hbm.at[idx_all.at[pl.ds(c * CT, CT)]],
                             xb, sem)
            pltpu.async_copy(pt_hbm.at[pl.ds(s_lo + c * SCH, SCH)], pb, sem)

        def wait_in(bi):
            xb, ob, pb, sem, osem = bufs[bi]
            pltpu.make_async_copy(
                wt_hbm.at[idx_all.at[pl.ds(0, CT)]], xb, sem).wait()
            pltpu.make_async_copy(pt_hbm.at[pl.ds(0, SCH)], pb, sem).wait()

        def fire_out(c, bi):
            xb, ob, pb, sem, osem = bufs[bi]
            s0 = s_lo + c * SCH
            for b in range(B):
                pltpu.async_copy(ob.at[pl.ds(b * SCH, SCH)],
                                 out_hbm.at[b, pl.ds(s0, SCH)], osem)

        def drain_out(bi):
            xb, ob, pb, sem, osem = bufs[bi]
            for b in range(B):
                pltpu.make_async_copy(ob.at[pl.ds(b * SCH, SCH)],
                                      out_hbm.at[b, pl.ds(0, SCH)],
                                      osem).wait()

        def compute(bi):
            xb, ob, pb, sem, osem = bufs[bi]
            zero = jnp.zeros((LANES,), jnp.float32)
            for g in range(CT // GT):
                t0 = g * GT

                def p_add(j, carry):
                    sl = pl.ds(j * LANES, LANES)
                    out = []
                    for t in range(GT):
                        a, q = carry[2 * t], carry[2 * t + 1]
                        x = xb[t0 + t, sl] + pb[t, sl]
                        xb[t0 + t, sl] = x
                        out.append(a + x)
                        out.append(q + x * x)
                    return tuple(out)

                accs = lax.fori_loop(0, NV, p_add, (zero,) * (2 * GT))

                stats = []
                for t in range(GT):
                    mean = _lane_allsum(accs[2 * t]) * (1.0 / D)
                    var = _lane_allsum(accs[2 * t + 1]) * (1.0 / D) \
                        - mean * mean
                    stats.append(mean)
                    stats.append(_rsqrt_vec(var + EPS))

                def p_norm(j, carry):
                    sl = pl.ds(j * LANES, LANES)
                    gj = g_v[sl]
                    bj = b_v[sl]
                    for t in range(GT):
                        x = xb[t0 + t, sl]
                        y = ((x - carry[2 * t]) * carry[2 * t + 1]) * gj + bj
                        ob[t0 + t, sl] = y
                    return carry

                lax.fori_loop(0, NV, p_norm, tuple(stats))

        def step(c, bi):
            wait_in(bi)

            @pl.when(c >= 2)
            def _():
                drain_out(bi)

            compute(bi)
            fire_out(c, bi)

            @pl.when(c + 2 < n_chunks)
            def _():
                issue(c + 2, bi)

        issue(0, 0)
        issue(1, 1)

        def pair_body(p, _):
            step(2 * p, 0)
            step(2 * p + 1, 1)
            return 0

        lax.fori_loop(0, n_pairs, pair_body, 0)
        drain_out(0)
        drain_out(1)

    return embed_ln


def kernel(input_ids, word_table, pos_table, gamma, beta):
    B, S = input_ids.shape
    V, D = word_table.shape
    info = plsc.get_sparse_core_info()
    NW = info.num_cores * info.num_subcores
    s_per_w = S // NW
    n_chunks = s_per_w // SCH
    ids_perm = (input_ids.astype(jnp.int32)
                .reshape(B, NW, n_chunks, SCH)
                .transpose(1, 2, 0, 3)
                .reshape(B * S))
    fn = _build(B, S, D)
    return fn(ids_perm, word_table, pos_table, gamma, beta)


# speedup vs baseline: 1.0054x; 1.0054x over previous
import functools

import jax
import jax.numpy as jnp
from jax import lax
from jax.experimental import pallas as pl
from jax.experimental.pallas import tpu as pltpu
from jax.experimental.pallas import tpu_sc as plsc

EPS = 1e-12
LANES = 16
GT = 8
SCH = 8

_GATHER_DNUMS = lax.GatherDimensionNumbers(
    offset_dims=(), collapsed_slice_dims=(0,), start_index_map=(0,))


def _lane_rotate(x, k):
    idx = jnp.bitwise_and(lax.iota(jnp.int32, LANES) + k, LANES - 1)
    return lax.gather(x, idx[:, None], _GATHER_DNUMS, slice_sizes=(1,),
                      mode=lax.GatherScatterMode.PROMISE_IN_BOUNDS)


def _lane_allsum(x):
    for k in (8, 4, 2, 1):
        x = x + _lane_rotate(x, k)
    return x


def _rsqrt_vec(x):
    i = lax.bitcast_convert_type(x, jnp.int32)
    i = jnp.int32(0x5F3759DF) - lax.shift_right_logical(i, 1)
    y = lax.bitcast_convert_type(i, jnp.float32)
    for _ in range(3):
        y = y * (1.5 - 0.5 * x * y * y)
    return y


@functools.lru_cache(maxsize=None)
def _build(B, S, D):
    info = plsc.get_sparse_core_info()
    NC, NS = info.num_cores, info.num_subcores
    NW = NC * NS
    T = B * S
    per_w = T // NW
    s_per_w = S // NW
    n_chunks = s_per_w // SCH
    CT = B * SCH
    NV = D // LANES
    n_pairs = n_chunks // 2

    mesh = plsc.VectorSubcoreMesh(core_axis_name="c", subcore_axis_name="s")

    @functools.partial(
        pl.kernel,
        mesh=mesh,
        out_type=jax.ShapeDtypeStruct((B, S, D), jnp.float32),
        scratch_types=[
            pltpu.VMEM((per_w,), jnp.int32),
            pltpu.VMEM((CT, D), jnp.float32),
            pltpu.VMEM((CT, D), jnp.float32),
            pltpu.VMEM((B, SCH, D), jnp.float32),
            pltpu.VMEM((B, SCH, D), jnp.float32),
            pltpu.VMEM((SCH, D), jnp.float32),
            pltpu.VMEM((SCH, D), jnp.float32),
            pltpu.VMEM((D,), jnp.float32),
            pltpu.VMEM((D,), jnp.float32),
            pltpu.SemaphoreType.DMA,
            pltpu.SemaphoreType.DMA,
            pltpu.SemaphoreType.DMA,
            pltpu.SemaphoreType.DMA,
        ],
    )
    def embed_ln(ids_hbm, wt_hbm, pt_hbm, g_hbm, b_hbm, out_hbm,
                 idx_all, xb0, xb1, ob0, ob1, pb0, pb1, g_v, b_v,
                 sem0, sem1, osem0, osem1):
        wid = lax.axis_index("s") * NC + lax.axis_index("c")
        s_lo = wid * s_per_w
        pltpu.sync_copy(g_hbm, g_v)
        pltpu.sync_copy(b_hbm, b_v)
        pltpu.sync_copy(ids_hbm.at[pl.ds(wid * per_w, per_w)], idx_all)

        bufs = ((xb0, ob0, pb0, sem0, osem0), (xb1, ob1, pb1, sem1, osem1))

        def issue(c, bi):
            xb, ob, pb, sem, osem = bufs[bi]
            pltpu.async_copy(wt_hbm.at[idx_all.at[pl.ds(c * CT, CT)]],
                             xb, sem)
            pltpu.async_copy(pt_hbm.at[pl.ds(s_lo + c * SCH, SCH)], pb, sem)

        def wait_in(bi):
            xb, ob, pb, sem, osem = bufs[bi]
            pltpu.make_async_copy(
                wt_hbm.at[idx_all.at[pl.ds(0, CT)]], xb, sem).wait()
            pltpu.make_async_copy(pt_hbm.at[pl.ds(0, SCH)], pb, sem).wait()

        def fire_out(c, bi):
            xb, ob, pb, sem, osem = bufs[bi]
            s0 = s_lo + c * SCH
            pltpu.async_copy(ob, out_hbm.at[:, pl.ds(s0, SCH)], osem)

        def drain_out(bi):
            xb, ob, pb, sem, osem = bufs[bi]
            pltpu.make_async_copy(ob, out_hbm.at[:, pl.ds(0, SCH)],
                                  osem).wait()

        def compute(bi):
            xb, ob, pb, sem, osem = bufs[bi]
            zero = jnp.zeros((LANES,), jnp.float32)
            for g in range(CT // GT):
                t0 = g * GT

                def p_add(j, carry):
                    sl = pl.ds(j * LANES, LANES)
                    out = []
                    for t in range(GT):
                        a, q = carry[2 * t], carry[2 * t + 1]
                        x = xb[t0 + t, sl] + pb[t, sl]
                        xb[t0 + t, sl] = x
                        out.append(a + x)
                        out.append(q + x * x)
                    return tuple(out)

                accs = lax.fori_loop(0, NV, p_add, (zero,) * (2 * GT))

                stats = []
                for t in range(GT):
                    mean = _lane_allsum(accs[2 * t]) * (1.0 / D)
                    var = _lane_allsum(accs[2 * t + 1]) * (1.0 / D) \
                        - mean * mean
                    stats.append(mean)
                    stats.append(_rsqrt_vec(var + EPS))

                def p_norm(j, carry):
                    sl = pl.ds(j * LANES, LANES)
                    gj = g_v[sl]
                    bj = b_v[sl]
                    for t in range(GT):
                        x = xb[t0 + t, sl]
                        y = ((x - carry[2 * t]) * carry[2 * t + 1]) * gj + bj
                        ob[g, t, sl] = y
                    return carry

                lax.fori_loop(0, NV, p_norm, tuple(stats))

        def step(c, bi):
            wait_in(bi)

            @pl.when(c >= 2)
            def _():
                drain_out(bi)

            compute(bi)
            fire_out(c, bi)

            @pl.when(c + 2 < n_chunks)
            def _():
                issue(c + 2, bi)

        issue(0, 0)
        issue(1, 1)

        def pair_body(p, _):
            step(2 * p, 0)
            step(2 * p + 1, 1)
            return 0

        lax.fori_loop(0, n_pairs, pair_body, 0)
        drain_out(0)
        drain_out(1)

    return embed_ln


def kernel(input_ids, word_table, pos_table, gamma, beta):
    B, S = input_ids.shape
    V, D = word_table.shape
    info = plsc.get_sparse_core_info()
    NW = info.num_cores * info.num_subcores
    s_per_w = S // NW
    n_chunks = s_per_w // SCH
    ids_perm = (input_ids.astype(jnp.int32)
                .reshape(B, NW, n_chunks, SCH)
                .transpose(1, 2, 0, 3)
                .reshape(B * S))
    fn = _build(B, S, D)
    return fn(ids_perm, word_table, pos_table, gamma, beta)


# speedup vs baseline: 1.2186x; 1.2120x over previous
import functools

import jax
import jax.numpy as jnp
from jax import lax
from jax.experimental import pallas as pl
from jax.experimental.pallas import tpu as pltpu
from jax.experimental.pallas import tpu_sc as plsc

EPS = 1e-12
LANES = 16
GT = 8
SCH = 8

_GATHER_DNUMS = lax.GatherDimensionNumbers(
    offset_dims=(), collapsed_slice_dims=(0,), start_index_map=(0,))


def _lane_rotate(x, k):
    idx = jnp.bitwise_and(lax.iota(jnp.int32, LANES) + k, LANES - 1)
    return lax.gather(x, idx[:, None], _GATHER_DNUMS, slice_sizes=(1,),
                      mode=lax.GatherScatterMode.PROMISE_IN_BOUNDS)


def _lane_allsum(x):
    for k in (8, 4, 2, 1):
        x = x + _lane_rotate(x, k)
    return x


def _rsqrt_vec(x):
    i = lax.bitcast_convert_type(x, jnp.int32)
    i = jnp.int32(0x5F3759DF) - lax.shift_right_logical(i, 1)
    y = lax.bitcast_convert_type(i, jnp.float32)
    for _ in range(3):
        y = y * (1.5 - 0.5 * x * y * y)
    return y


@functools.lru_cache(maxsize=None)
def _build(B, S, D):
    info = plsc.get_sparse_core_info()
    NC, NS = info.num_cores, info.num_subcores
    NW = NC * NS
    T = B * S
    per_w = T // NW
    s_per_w = S // NW
    n_chunks = s_per_w // SCH
    CT = B * SCH
    NV = D // LANES
    n_tri = (n_chunks - 1) // 3

    mesh = plsc.VectorSubcoreMesh(core_axis_name="c", subcore_axis_name="s")

    @functools.partial(
        pl.kernel,
        mesh=mesh,
        out_type=jax.ShapeDtypeStruct((B, S, D), jnp.float32),
        scratch_types=[
            pltpu.VMEM((per_w,), jnp.int32),
            pltpu.VMEM((CT + SCH, D), jnp.float32),
            pltpu.VMEM((CT + SCH, D), jnp.float32),
            pltpu.VMEM((CT + SCH, D), jnp.float32),
            pltpu.VMEM((D,), jnp.float32),
            pltpu.VMEM((D,), jnp.float32),
            pltpu.SemaphoreType.DMA,
            pltpu.SemaphoreType.DMA,
            pltpu.SemaphoreType.DMA,
            pltpu.SemaphoreType.DMA,
            pltpu.SemaphoreType.DMA,
            pltpu.SemaphoreType.DMA,
        ],
    )
    def embed_ln(ids_hbm, wt_hbm, pt_hbm, g_hbm, b_hbm, out_hbm,
                 idx_all, xb0, xb1, xb2, g_v, b_v,
                 sem0, sem1, sem2, osem0, osem1, osem2):
        wid = lax.axis_index("s") * NC + lax.axis_index("c")
        s_lo = wid * s_per_w
        pltpu.sync_copy(g_hbm, g_v)
        pltpu.sync_copy(b_hbm, b_v)
        pltpu.sync_copy(ids_hbm.at[pl.ds(wid * per_w, per_w)], idx_all)

        xbs = (xb0, xb1, xb2)
        sems = (sem0, sem1, sem2)
        osems = (osem0, osem1, osem2)

        def issue(c, bi):
            xb, sem = xbs[bi], sems[bi]
            pltpu.async_copy(wt_hbm.at[idx_all.at[pl.ds(c * CT, CT)]],
                             xb.at[pl.ds(0, CT)], sem)
            pltpu.async_copy(pt_hbm.at[pl.ds(s_lo + c * SCH, SCH)],
                             xb.at[pl.ds(CT, SCH)], sem)

        def wait_in(bi):
            xb, sem = xbs[bi], sems[bi]
            pltpu.make_async_copy(wt_hbm.at[idx_all.at[pl.ds(0, CT)]],
                                  xb.at[pl.ds(0, CT)], sem).wait()
            pltpu.make_async_copy(pt_hbm.at[pl.ds(0, SCH)],
                                  xb.at[pl.ds(CT, SCH)], sem).wait()

        def fire_out(c, bi):
            xb, osem = xbs[bi], osems[bi]
            s0 = s_lo + c * SCH
            for b in range(B):
                pltpu.async_copy(xb.at[pl.ds(b * SCH, SCH)],
                                 out_hbm.at[b, pl.ds(s0, SCH)], osem)

        def drain_out(bi):
            xb, osem = xbs[bi], osems[bi]
            pltpu.make_async_copy(wt_hbm.at[pl.ds(0, CT)],
                                  xb.at[pl.ds(0, CT)], osem).wait()

        def compute(bi):
            xb = xbs[bi]
            zero = jnp.zeros((LANES,), jnp.float32)
            for g in range(CT // GT):
                t0 = g * GT

                def p_add(j, carry):
                    sl = pl.ds(j * LANES, LANES)
                    out = []
                    for t in range(GT):
                        a, q = carry[2 * t], carry[2 * t + 1]
                        x = xb[t0 + t, sl] + xb[CT + t, sl]
                        xb[t0 + t, sl] = x
                        out.append(a + x)
                        out.append(q + x * x)
                    return tuple(out)

                accs = lax.fori_loop(0, NV, p_add, (zero,) * (2 * GT))

                stats = []
                for t in range(GT):
                    mean = _lane_allsum(accs[2 * t]) * (1.0 / D)
                    var = _lane_allsum(accs[2 * t + 1]) * (1.0 / D) \
                        - mean * mean
                    stats.append(mean)
                    stats.append(_rsqrt_vec(var + EPS))

                def p_norm(j, carry):
                    sl = pl.ds(j * LANES, LANES)
                    gj = g_v[sl]
                    bj = b_v[sl]
                    for t in range(GT):
                        x = xb[t0 + t, sl]
                        y = ((x - carry[2 * t]) * carry[2 * t + 1]) * gj + bj
                        xb[t0 + t, sl] = y
                    return carry

                lax.fori_loop(0, NV, p_norm, tuple(stats))

        def step(c, k, issue_next=True):
            wait_in(k)
            compute(k)
            fire_out(c, k)
            nxt = (k + 2) % 3

            if issue_next:
                @pl.when(c + 2 < n_chunks)
                def _():
                    @pl.when(c >= 1)
                    def _():
                        drain_out(nxt)
                    issue(c + 2, nxt)

        issue(0, 0)
        issue(1, 1)

        def tri_body(p, _):
            c0 = 3 * p
            step(c0, 0)
            step(c0 + 1, 1)
            step(c0 + 2, 2)
            return 0

        lax.fori_loop(0, n_tri, tri_body, 0)
        step(n_chunks - 1, (n_chunks - 1) % 3, issue_next=False)
        drain_out((n_chunks - 3) % 3)
        drain_out((n_chunks - 2) % 3)
        drain_out((n_chunks - 1) % 3)

    return embed_ln


def kernel(input_ids, word_table, pos_table, gamma, beta):
    B, S = input_ids.shape
    V, D = word_table.shape
    info = plsc.get_sparse_core_info()
    NW = info.num_cores * info.num_subcores
    s_per_w = S // NW
    n_chunks = s_per_w // SCH
    ids_perm = (input_ids.astype(jnp.int32)
                .reshape(B, NW, n_chunks, SCH)
                .transpose(1, 2, 0, 3)
                .reshape(B * S))
    fn = _build(B, S, D)
    return fn(ids_perm, word_table, pos_table, gamma, beta)


# speedup vs baseline: 1.2297x; 1.0091x over previous
import functools

import jax
import jax.numpy as jnp
from jax import lax
from jax.experimental import pallas as pl
from jax.experimental.pallas import tpu as pltpu
from jax.experimental.pallas import tpu_sc as plsc

EPS = 1e-12
LANES = 16
GT = 8
SCH = 8

_GATHER_DNUMS = lax.GatherDimensionNumbers(
    offset_dims=(), collapsed_slice_dims=(0,), start_index_map=(0,))


def _lane_rotate(x, k):
    idx = jnp.bitwise_and(lax.iota(jnp.int32, LANES) + k, LANES - 1)
    return lax.gather(x, idx[:, None], _GATHER_DNUMS, slice_sizes=(1,),
                      mode=lax.GatherScatterMode.PROMISE_IN_BOUNDS)


def _lane_allsum(x):
    for k in (8, 4, 2, 1):
        x = x + _lane_rotate(x, k)
    return x


def _rsqrt_vec(x):
    i = lax.bitcast_convert_type(x, jnp.int32)
    i = jnp.int32(0x5F3759DF) - lax.shift_right_logical(i, 1)
    y = lax.bitcast_convert_type(i, jnp.float32)
    for _ in range(2):
        y = y * (1.5 - 0.5 * x * y * y)
    return y


@functools.lru_cache(maxsize=None)
def _build(B, S, D):
    info = plsc.get_sparse_core_info()
    NC, NS = info.num_cores, info.num_subcores
    NW = NC * NS
    T = B * S
    per_w = T // NW
    s_per_w = S // NW
    n_chunks = s_per_w // SCH
    CT = B * SCH
    NV = D // LANES
    n_tri = (n_chunks - 1) // 3

    mesh = plsc.VectorSubcoreMesh(core_axis_name="c", subcore_axis_name="s")

    @functools.partial(
        pl.kernel,
        mesh=mesh,
        out_type=jax.ShapeDtypeStruct((B, S, D), jnp.float32),
        scratch_types=[
            pltpu.VMEM((per_w,), jnp.int32),
            pltpu.VMEM((CT + SCH, D), jnp.float32),
            pltpu.VMEM((CT + SCH, D), jnp.float32),
            pltpu.VMEM((CT + SCH, D), jnp.float32),
            pltpu.VMEM((D,), jnp.float32),
            pltpu.VMEM((D,), jnp.float32),
            pltpu.SemaphoreType.DMA,
            pltpu.SemaphoreType.DMA,
            pltpu.SemaphoreType.DMA,
            pltpu.SemaphoreType.DMA,
            pltpu.SemaphoreType.DMA,
            pltpu.SemaphoreType.DMA,
        ],
    )
    def embed_ln(ids_hbm, wt_hbm, pt_hbm, g_hbm, b_hbm, out_hbm,
                 idx_all, xb0, xb1, xb2, g_v, b_v,
                 sem0, sem1, sem2, osem0, osem1, osem2):
        wid = lax.axis_index("s") * NC + lax.axis_index("c")
        s_lo = wid * s_per_w
        pltpu.sync_copy(g_hbm, g_v)
        pltpu.sync_copy(b_hbm, b_v)
        pltpu.sync_copy(ids_hbm.at[pl.ds(wid * per_w, per_w)], idx_all)

        xbs = (xb0, xb1, xb2)
        sems = (sem0, sem1, sem2)
        osems = (osem0, osem1, osem2)

        def issue(c, bi):
            xb, sem = xbs[bi], sems[bi]
            pltpu.async_copy(wt_hbm.at[idx_all.at[pl.ds(c * CT, CT)]],
                             xb.at[pl.ds(0, CT)], sem)
            pltpu.async_copy(pt_hbm.at[pl.ds(s_lo + c * SCH, SCH)],
                             xb.at[pl.ds(CT, SCH)], sem)

        def wait_in(bi):
            xb, sem = xbs[bi], sems[bi]
            pltpu.make_async_copy(wt_hbm.at[idx_all.at[pl.ds(0, CT)]],
                                  xb.at[pl.ds(0, CT)], sem).wait()
            pltpu.make_async_copy(pt_hbm.at[pl.ds(0, SCH)],
                                  xb.at[pl.ds(CT, SCH)], sem).wait()

        def fire_out(c, bi):
            xb, osem = xbs[bi], osems[bi]
            s0 = s_lo + c * SCH
            for b in range(B):
                pltpu.async_copy(xb.at[pl.ds(b * SCH, SCH)],
                                 out_hbm.at[b, pl.ds(s0, SCH)], osem)

        def drain_out(bi):
            xb, osem = xbs[bi], osems[bi]
            pltpu.make_async_copy(wt_hbm.at[pl.ds(0, CT)],
                                  xb.at[pl.ds(0, CT)], osem).wait()

        def compute(bi):
            xb = xbs[bi]
            zero = jnp.zeros((LANES,), jnp.float32)
            for g in range(CT // GT):
                t0 = g * GT

                def p_add(j, carry):
                    sl = pl.ds(j * LANES, LANES)
                    out = []
                    for t in range(GT):
                        a, q = carry[2 * t], carry[2 * t + 1]
                        x = xb[t0 + t, sl] + xb[CT + t, sl]
                        xb[t0 + t, sl] = x
                        out.append(a + x)
                        out.append(q + x * x)
                    return tuple(out)

                accs = lax.fori_loop(0, NV, p_add, (zero,) * (2 * GT))

                stats = []
                for t in range(GT):
                    mean = _lane_allsum(accs[2 * t]) * (1.0 / D)
                    var = _lane_allsum(accs[2 * t + 1]) * (1.0 / D) \
                        - mean * mean
                    stats.append(mean)
                    stats.append(_rsqrt_vec(var + EPS))

                def p_norm(j, carry):
                    sl = pl.ds(j * LANES, LANES)
                    gj = g_v[sl]
                    bj = b_v[sl]
                    for t in range(GT):
                        x = xb[t0 + t, sl]
                        y = ((x - carry[2 * t]) * carry[2 * t + 1]) * gj + bj
                        xb[t0 + t, sl] = y
                    return carry

                lax.fori_loop(0, NV, p_norm, tuple(stats))

        def step(c, k, issue_next=True):
            wait_in(k)
            compute(k)
            fire_out(c, k)
            nxt = (k + 2) % 3

            if issue_next:
                @pl.when(c + 2 < n_chunks)
                def _():
                    @pl.when(c >= 1)
                    def _():
                        drain_out(nxt)
                    issue(c + 2, nxt)

        issue(0, 0)
        issue(1, 1)

        def tri_body(p, _):
            c0 = 3 * p
            step(c0, 0)
            step(c0 + 1, 1)
            step(c0 + 2, 2)
            return 0

        lax.fori_loop(0, n_tri, tri_body, 0)
        step(n_chunks - 1, (n_chunks - 1) % 3, issue_next=False)
        drain_out((n_chunks - 3) % 3)
        drain_out((n_chunks - 2) % 3)
        drain_out((n_chunks - 1) % 3)

    return embed_ln


def kernel(input_ids, word_table, pos_table, gamma, beta):
    B, S = input_ids.shape
    V, D = word_table.shape
    info = plsc.get_sparse_core_info()
    NW = info.num_cores * info.num_subcores
    s_per_w = S // NW
    n_chunks = s_per_w // SCH
    ids_perm = (input_ids.astype(jnp.int32)
                .reshape(B, NW, n_chunks, SCH)
                .transpose(1, 2, 0, 3)
                .reshape(B * S))
    fn = _build(B, S, D)
    return fn(ids_perm, word_table, pos_table, gamma, beta)
